# initial kernel scaffold (unmeasured)
import jax
import jax.numpy as jnp
from jax import lax
from jax.experimental import pallas as pl
from jax.experimental.pallas import tpu as pltpu


def kernel(
    t,
):
    def body(*refs):
        pass

    out_shape = jax.ShapeDtypeStruct(..., jnp.float32)
    return pl.pallas_call(body, out_shape=out_shape)(...)


# baseline (device time: 194024 ns/iter reference)
import jax
import jax.numpy as jnp
from jax import lax
from jax.experimental import pallas as pl
from jax.experimental.pallas import tpu as pltpu

N_DEV = 8


def kernel(t):
    m_per, n = t.shape
    c = m_per // N_DEV

    def body(
        x_ref,
        out_ref,
        rs_send,
        rs_recv,
        ag_recv,
        f_buf,
        rs_send_sems,
        rs_recv_sems,
        ag_send_sems,
        ag_recv_sems,
    ):
        my = lax.axis_index("i")
        left = (my + N_DEV - 1) % N_DEV
        right = (my + 1) % N_DEV

        barrier_sem = pltpu.get_barrier_semaphore()
        for nbr in (left, right):
            pl.semaphore_signal(
                barrier_sem,
                inc=1,
                device_id=(nbr,),
                device_id_type=pl.DeviceIdType.MESH,
            )
        pl.semaphore_wait(barrier_sem, 2)

        c_send0 = my
        rs_send[0] = x_ref[pl.ds(c_send0 * c, c), :]
        for h in range(N_DEV - 1):
            rdma = pltpu.make_async_remote_copy(
                src_ref=rs_send.at[h],
                dst_ref=rs_recv.at[h],
                send_sem=rs_send_sems.at[h],
                recv_sem=rs_recv_sems.at[h],
                device_id=(right,),
                device_id_type=pl.DeviceIdType.MESH,
            )
            rdma.start()
            rdma.wait()

            c_recv = (my + 2 * N_DEV - h - 1) % N_DEV
            val = rs_recv[h] + x_ref[pl.ds(c_recv * c, c), :]
            if h < N_DEV - 2:
                rs_send[h + 1] = val
            else:
                r = jnp.maximum(val, 0.0)
                fval = jnp.tanh(val) * val * val + r * r * r
                f_buf[...] = fval
                out_ref[pl.ds(c_recv * c, c), :] = fval

        for g in range(N_DEV - 1):
            src = f_buf if g == 0 else ag_recv.at[g - 1]
            rdma = pltpu.make_async_remote_copy(
                src_ref=src,
                dst_ref=ag_recv.at[g],
                send_sem=ag_send_sems.at[g],
                recv_sem=ag_recv_sems.at[g],
                device_id=(right,),
                device_id_type=pl.DeviceIdType.MESH,
            )
            rdma.start()
            rdma.wait()

            c_recv = (my + N_DEV - g) % N_DEV
            out_ref[pl.ds(c_recv * c, c), :] = ag_recv[g]

    return pl.pallas_call(
        body,
        out_shape=jax.ShapeDtypeStruct((m_per, n), t.dtype),
        in_specs=[pl.BlockSpec(memory_space=pltpu.VMEM)],
        out_specs=pl.BlockSpec(memory_space=pltpu.VMEM),
        scratch_shapes=[
            pltpu.VMEM((N_DEV - 1, c, n), t.dtype),
            pltpu.VMEM((N_DEV - 1, c, n), t.dtype),
            pltpu.VMEM((N_DEV - 1, c, n), t.dtype),
            pltpu.VMEM((c, n), t.dtype),
            pltpu.SemaphoreType.DMA((N_DEV - 1,)),
            pltpu.SemaphoreType.DMA((N_DEV - 1,)),
            pltpu.SemaphoreType.DMA((N_DEV - 1,)),
            pltpu.SemaphoreType.DMA((N_DEV - 1,)),
        ],
        compiler_params=pltpu.CompilerParams(collective_id=0),
    )(t)


# device time: 78527 ns/iter; 2.4708x vs baseline; 2.4708x over previous
import jax
import jax.numpy as jnp
from jax import lax
from jax.experimental import pallas as pl
from jax.experimental.pallas import tpu as pltpu

N_DEV = 8

AXIS_MASK = {"x": 1, "y": 3, "z": 4}

PARTS = (
    (0, 704, ("x", "y", "z")),
    (704, 704, ("y", "z", "x")),
    (1408, 640, ("z", "x", "y")),
)


def kernel(t):
    m_per, n = t.shape
    assert m_per == sum(pr[1] for pr in PARTS)

    rs_sizes = [[rows >> (ph + 1) for ph in range(3)] for _, rows, _ in PARTS]
    ag_sizes = [[rows >> (3 - ph) for ph in range(3)] for _, rows, _ in PARTS]
    rs_offs = [[sum(s[:ph]) for ph in range(3)] for s in rs_sizes]
    ag_offs = [[sum(s[:ph]) for ph in range(3)] for s in ag_sizes]

    def body(
        x_ref,
        out_ref,
        sbuf0, sbuf1, sbuf2,
        rbuf0, rbuf1, rbuf2,
        abuf0, abuf1, abuf2,
        send_sems,
        recv_sems,
    ):
        sbufs = [sbuf0, sbuf1, sbuf2]
        rbufs = [rbuf0, rbuf1, rbuf2]
        abufs = [abuf0, abuf1, abuf2]

        p = lax.axis_index("i")
        bit = {
            "x": ((p ^ (p >> 1)) & 1) == 1,
            "y": ((p >> 1) & 1) == 1,
            "z": ((p >> 2) & 1) == 1,
        }

        barrier_sem = pltpu.get_barrier_semaphore()
        for m in (1, 3, 4):
            pl.semaphore_signal(
                barrier_sem,
                inc=1,
                device_id=(p ^ m,),
                device_id_type=pl.DeviceIdType.MESH,
            )
        pl.semaphore_wait(barrier_sem, 3)

        vals = [
            x_ref[pl.ds(base, rows), :] for base, rows, _ in PARTS
        ]

        for ph in range(3):
            started = []
            for pi, (base, rows, order) in enumerate(PARTS):
                axis = order[ph]
                half = rs_sizes[pi][ph]
                b = bit[axis]
                v = vals[pi]
                lower, upper = v[:half, :], v[half:, :]
                sbufs[pi][pl.ds(0, half), :] = jnp.where(b, lower, upper)
                keep = jnp.where(b, upper, lower)
                rdma = pltpu.make_async_remote_copy(
                    src_ref=sbufs[pi].at[pl.ds(0, half), :],
                    dst_ref=rbufs[pi].at[pl.ds(rs_offs[pi][ph], half), :],
                    send_sem=send_sems.at[pi, ph],
                    recv_sem=recv_sems.at[pi, ph],
                    device_id=(p ^ AXIS_MASK[axis],),
                    device_id_type=pl.DeviceIdType.MESH,
                )
                rdma.start()
                started.append((rdma, keep, pi, half))
            for rdma, keep, pi, half in started:
                rdma.wait()
                vals[pi] = keep + rbufs[pi][pl.ds(rs_offs[pi][ph], half), :]

        gs = []
        for pi in range(3):
            s = vals[pi]
            r = jnp.maximum(s, 0.0)
            gs.append(jnp.tanh(s) * s * s + r * r * r)

        for ph in range(3):
            started = []
            for pi, (base, rows, order) in enumerate(PARTS):
                axis = order[2 - ph]
                cur = ag_sizes[pi][ph]
                sbufs[pi][pl.ds(0, cur), :] = gs[pi]
                rdma = pltpu.make_async_remote_copy(
                    src_ref=sbufs[pi].at[pl.ds(0, cur), :],
                    dst_ref=abufs[pi].at[pl.ds(ag_offs[pi][ph], cur), :],
                    send_sem=send_sems.at[pi, 3 + ph],
                    recv_sem=recv_sems.at[pi, 3 + ph],
                    device_id=(p ^ AXIS_MASK[axis],),
                    device_id_type=pl.DeviceIdType.MESH,
                )
                rdma.start()
                started.append((rdma, pi, axis, cur))
            for rdma, pi, axis, cur in started:
                rdma.wait()
                recv = abufs[pi][pl.ds(ag_offs[pi][ph], cur), :]
                g = gs[pi]
                gs[pi] = jnp.where(
                    bit[axis],
                    jnp.concatenate([recv, g], axis=0),
                    jnp.concatenate([g, recv], axis=0),
                )

        for pi, (base, rows, _) in enumerate(PARTS):
            out_ref[pl.ds(base, rows), :] = gs[pi]

    scratch = []
    for _, rows, _ in PARTS:
        scratch.append(pltpu.VMEM((rows // 2, n), t.dtype))
    for pi in range(3):
        scratch.append(pltpu.VMEM((sum(rs_sizes[pi]), n), t.dtype))
    for pi in range(3):
        scratch.append(pltpu.VMEM((sum(ag_sizes[pi]), n), t.dtype))
    scratch.append(pltpu.SemaphoreType.DMA((3, 6)))
    scratch.append(pltpu.SemaphoreType.DMA((3, 6)))

    return pl.pallas_call(
        body,
        out_shape=jax.ShapeDtypeStruct((m_per, n), t.dtype),
        in_specs=[pl.BlockSpec(memory_space=pltpu.VMEM)],
        out_specs=pl.BlockSpec(memory_space=pltpu.VMEM),
        scratch_shapes=scratch,
        compiler_params=pltpu.CompilerParams(collective_id=0),
    )(t)


# device time: 76332 ns/iter; 2.5418x vs baseline; 1.0288x over previous
import jax
import jax.numpy as jnp
from jax import lax
from jax.experimental import pallas as pl
from jax.experimental.pallas import tpu as pltpu

N_DEV = 8

AXIS_MASK = {"x": 1, "y": 3, "z": 4}

PARTS = (
    (0, 704, ("x", "y", "z")),
    (704, 704, ("y", "z", "x")),
    (1408, 640, ("z", "x", "y")),
)


def kernel(t):
    m_per, n = t.shape
    assert m_per == sum(pr[1] for pr in PARTS)

    rs_sizes = [[rows >> (ph + 1) for ph in range(3)] for _, rows, _ in PARTS]
    rs_offs = [[sum(s[:ph]) for ph in range(3)] for s in rs_sizes]

    def body(
        x_ref,
        out_ref,
        rbuf0, rbuf1, rbuf2,
        acc0, acc1, acc2,
        send_sems,
        recv_sems,
    ):
        rbufs = [rbuf0, rbuf1, rbuf2]
        accs = [acc0, acc1, acc2]

        p = lax.axis_index("i")
        bit = {
            "x": (p ^ (p >> 1)) & 1,
            "y": (p >> 1) & 1,
            "z": (p >> 2) & 1,
        }

        barrier_sem = pltpu.get_barrier_semaphore()
        for m in (1, 3, 4):
            pl.semaphore_signal(
                barrier_sem,
                inc=1,
                device_id=(p ^ m,),
                device_id_type=pl.DeviceIdType.MESH,
            )
        pl.semaphore_wait(barrier_sem, 3)

        los = [None, None, None]
        for ph in range(3):
            started = []
            for pi, (base, rows, order) in enumerate(PARTS):
                axis = order[ph]
                half = rs_sizes[pi][ph]
                b = bit[axis]
                if ph == 0:
                    src = x_ref.at[pl.ds(base + (1 - b) * half, half), :]
                else:
                    src = accs[pi].at[pl.ds((1 - b) * half, half), :]
                rdma = pltpu.make_async_remote_copy(
                    src_ref=src,
                    dst_ref=rbufs[pi].at[pl.ds(rs_offs[pi][ph], half), :],
                    send_sem=send_sems.at[pi, ph],
                    recv_sem=recv_sems.at[pi, ph],
                    device_id=(p ^ AXIS_MASK[axis],),
                    device_id_type=pl.DeviceIdType.MESH,
                )
                rdma.start()
                started.append((rdma, pi, b, half))
            for rdma, pi, b, half in started:
                base, rows, order = PARTS[pi]
                rdma.wait()
                recv = rbufs[pi][pl.ds(rs_offs[pi][ph], half), :]
                if ph == 0:
                    keep = x_ref[pl.ds(base + b * half, half), :]
                    los[pi] = base + b * half
                else:
                    keep = accs[pi][pl.ds(b * half, half), :]
                    los[pi] = los[pi] + b * half
                accs[pi][pl.ds(0, half), :] = keep + recv

        for pi, (base, rows, _) in enumerate(PARTS):
            chunk = rows >> 3
            s = accs[pi][pl.ds(0, chunk), :]
            r = jnp.maximum(s, 0.0)
            out_ref[pl.ds(los[pi], chunk), :] = jnp.tanh(s) * s * s + r * r * r

        for ph in range(3):
            started = []
            for pi, (base, rows, order) in enumerate(PARTS):
                axis = order[2 - ph]
                cur = rows >> (3 - ph)
                rdma = pltpu.make_async_remote_copy(
                    src_ref=out_ref.at[pl.ds(los[pi], cur), :],
                    dst_ref=out_ref.at[pl.ds(los[pi], cur), :],
                    send_sem=send_sems.at[pi, 3 + ph],
                    recv_sem=recv_sems.at[pi, 3 + ph],
                    device_id=(p ^ AXIS_MASK[axis],),
                    device_id_type=pl.DeviceIdType.MESH,
                )
                rdma.start()
                started.append((rdma, pi, axis, cur))
            for rdma, pi, axis, cur in started:
                rdma.wait()
                los[pi] = los[pi] - bit[axis] * cur

    scratch = []
    for pi in range(3):
        scratch.append(pltpu.VMEM((sum(rs_sizes[pi]), n), t.dtype))
    for _, rows, _ in PARTS:
        scratch.append(pltpu.VMEM((rows // 2, n), t.dtype))
    scratch.append(pltpu.SemaphoreType.DMA((3, 6)))
    scratch.append(pltpu.SemaphoreType.DMA((3, 6)))

    return pl.pallas_call(
        body,
        out_shape=jax.ShapeDtypeStruct((m_per, n), t.dtype),
        in_specs=[pl.BlockSpec(memory_space=pltpu.VMEM)],
        out_specs=pl.BlockSpec(memory_space=pltpu.VMEM),
        scratch_shapes=scratch,
        compiler_params=pltpu.CompilerParams(collective_id=0),
    )(t)


# device time: 75574 ns/iter; 2.5673x vs baseline; 1.0100x over previous
import jax
import jax.numpy as jnp
from jax import lax
from jax.experimental import pallas as pl
from jax.experimental.pallas import tpu as pltpu

N_DEV = 8

AXIS_MASK = {"x": 1, "y": 3, "z": 4}

PARTS = (
    (0, 704, ("x", "y", "z")),
    (704, 704, ("y", "z", "x")),
    (1408, 640, ("z", "x", "y")),
)


def kernel(t):
    m_per, n = t.shape
    assert m_per == sum(pr[1] for pr in PARTS)

    rs_sizes = [[rows >> (ph + 1) for ph in range(3)] for _, rows, _ in PARTS]
    rs_offs = [[sum(s[:ph]) for ph in range(3)] for s in rs_sizes]

    def body(
        x_ref,
        out_ref,
        rbuf0, rbuf1, rbuf2,
        acc0, acc1, acc2,
        send_sems,
        recv_sems,
    ):
        rbufs = [rbuf0, rbuf1, rbuf2]
        accs = [acc0, acc1, acc2]

        p = lax.axis_index("i")
        bit = {
            "x": (p ^ (p >> 1)) & 1,
            "y": (p >> 1) & 1,
            "z": (p >> 2) & 1,
        }

        barrier_sem = pltpu.get_barrier_semaphore()
        for m in (1, 3, 4):
            pl.semaphore_signal(
                barrier_sem,
                inc=1,
                device_id=(p ^ m,),
                device_id_type=pl.DeviceIdType.MESH,
            )
        pl.semaphore_wait(barrier_sem, 3)

        los = [None, None, None]

        def start_rs(pi, ph):
            base, rows, order = PARTS[pi]
            axis = order[ph]
            half = rs_sizes[pi][ph]
            b = bit[axis]
            if ph == 0:
                src = x_ref.at[pl.ds(base + (1 - b) * half, half), :]
            else:
                src = accs[pi].at[pl.ds((1 - b) * half, half), :]
            rdma = pltpu.make_async_remote_copy(
                src_ref=src,
                dst_ref=rbufs[pi].at[pl.ds(rs_offs[pi][ph], half), :],
                send_sem=send_sems.at[pi, ph],
                recv_sem=recv_sems.at[pi, ph],
                device_id=(p ^ AXIS_MASK[axis],),
                device_id_type=pl.DeviceIdType.MESH,
            )
            rdma.start()
            return rdma

        def start_ag(pi, ph):
            _, rows, order = PARTS[pi]
            axis = order[2 - ph]
            cur = rows >> (3 - ph)
            rdma = pltpu.make_async_remote_copy(
                src_ref=out_ref.at[pl.ds(los[pi], cur), :],
                dst_ref=out_ref.at[pl.ds(los[pi], cur), :],
                send_sem=send_sems.at[pi, 3 + ph],
                recv_sem=recv_sems.at[pi, 3 + ph],
                device_id=(p ^ AXIS_MASK[axis],),
                device_id_type=pl.DeviceIdType.MESH,
            )
            rdma.start()
            return rdma

        inflight = [start_rs(pi, 0) for pi in range(3)]
        for ph in range(3):
            nxt = [None, None, None]
            for pi, (base, rows, order) in enumerate(PARTS):
                half = rs_sizes[pi][ph]
                b = bit[order[ph]]
                inflight[pi].wait()
                recv = rbufs[pi][pl.ds(rs_offs[pi][ph], half), :]
                if ph == 0:
                    keep = x_ref[pl.ds(base + b * half, half), :]
                    los[pi] = base + b * half
                else:
                    keep = accs[pi][pl.ds(b * half, half), :]
                    los[pi] = los[pi] + b * half
                if ph < 2:
                    accs[pi][pl.ds(0, half), :] = keep + recv
                    nxt[pi] = start_rs(pi, ph + 1)
                else:
                    s = keep + recv
                    r = jnp.maximum(s, 0.0)
                    out_ref[pl.ds(los[pi], half), :] = (
                        jnp.tanh(s) * s * s + r * r * r
                    )
                    nxt[pi] = start_ag(pi, 0)
            inflight = nxt

        for ph in range(3):
            nxt = [None, None, None]
            for pi, (base, rows, order) in enumerate(PARTS):
                axis = order[2 - ph]
                cur = rows >> (3 - ph)
                inflight[pi].wait()
                los[pi] = los[pi] - bit[axis] * cur
                if ph < 2:
                    nxt[pi] = start_ag(pi, ph + 1)
            inflight = nxt

    scratch = []
    for pi in range(3):
        scratch.append(pltpu.VMEM((sum(rs_sizes[pi]), n), t.dtype))
    for _, rows, _ in PARTS:
        scratch.append(pltpu.VMEM((rows // 2, n), t.dtype))
    scratch.append(pltpu.SemaphoreType.DMA((3, 6)))
    scratch.append(pltpu.SemaphoreType.DMA((3, 6)))

    return pl.pallas_call(
        body,
        out_shape=jax.ShapeDtypeStruct((m_per, n), t.dtype),
        in_specs=[pl.BlockSpec(memory_space=pltpu.VMEM)],
        out_specs=pl.BlockSpec(memory_space=pltpu.VMEM),
        scratch_shapes=scratch,
        compiler_params=pltpu.CompilerParams(collective_id=0),
    )(t)
